# baseline (speedup 1.0000x reference)
import dataclasses
import functools

import jax
import jax.numpy as jnp
import numpy as np
from jax import lax
from jax.experimental import pallas as pl
from jax.experimental.pallas import tpu as pltpu
from jax.experimental.pallas import tpu_sc as plsc

N = 10000
E = 320000
D = 128
H = 8
C = 128
HC = H * C
HID = 512

NW = 32
RNG = 320
SUB = 40
NSUB = RNG // SUB
NBUF = 4
NPAD = NW * RNG
CH = 2048
EP = E + N
NCHUNK = -(-EP // CH)
EPAD = NCHUNK * CH
KCAP = 12288
SCAP = 2048
G = 8
XW = HC + 128
_SPLAT_IDX = tuple(np.full((16,), h, np.int32) for h in range(H))


def _sc_compiler_params():
    cp = pltpu.CompilerParams()
    if "needs_layout_passes" in pltpu.CompilerParams.__dataclass_fields__:
        cp = dataclasses.replace(cp, needs_layout_passes=False)
    return cp


def _prep_body(x_ref, wg_ref, asrc_ref, adst_ref, xpw_ref, ad_ref):
    xb = x_ref[...]
    wg = wg_ref[...]
    xp = lax.dot_general(xb, wg, (((1,), (1,)), ((), ())),
                         preferred_element_type=jnp.float32)
    xpw_ref[:, :HC] = xp
    z8 = jnp.zeros((xb.shape[0], 8), jnp.float32)
    z112 = jnp.zeros((xb.shape[0], 112), jnp.float32)
    a_s = []
    a_d = []
    for h in range(H):
        blk = xp[:, h * C:(h + 1) * C]
        a_s.append(jnp.sum(blk * asrc_ref[h:h + 1, :], axis=1, keepdims=True))
        a_d.append(jnp.sum(blk * adst_ref[h:h + 1, :], axis=1, keepdims=True))
    xpw_ref[:, HC:] = jnp.concatenate(a_s + [z8, z112], axis=1)
    ad_ref[...] = jnp.concatenate(a_d + [z8], axis=1)


def _prep(x_pad, w_gat, asrc, adst):
    BR = 1024
    grid = (NPAD // BR,)
    return pl.pallas_call(
        _prep_body,
        grid=grid,
        in_specs=[
            pl.BlockSpec((BR, D), lambda i: (i, 0)),
            pl.BlockSpec((HC, D), lambda i: (0, 0)),
            pl.BlockSpec((H, C), lambda i: (0, 0)),
            pl.BlockSpec((H, C), lambda i: (0, 0)),
        ],
        out_specs=[
            pl.BlockSpec((BR, XW), lambda i: (i, 0)),
            pl.BlockSpec((BR, 16), lambda i: (i, 0)),
        ],
        out_shape=[
            jax.ShapeDtypeStruct((NPAD, XW), jnp.float32),
            jax.ShapeDtypeStruct((NPAD, 16), jnp.float32),
        ],
    )(x_pad, w_gat, asrc, adst)


def _edge_body(dst_hbm, src_hbm, ad_hbm, xpw_hbm, zer_hbm,
               acc_hbm, den_hbm,
               ad_tab, dst_buf, src_buf, dl_list, dls_buf, sls_buf,
               rowbuf, bsbuf, acc, den, sem_a, sem_b, sem_c, sem_d):
    cid = lax.axis_index("c")
    sid = lax.axis_index("s")
    w = sid * 2 + cid
    lo = w * RNG
    sems = (sem_a, sem_b, sem_c, sem_d)

    pltpu.sync_copy(ad_hbm.at[pl.ds(lo * 16, RNG * 16)],
                    ad_tab.at[pl.ds(0, RNG * 16)])

    def fire_chunk(ci, b):
        pltpu.async_copy(dst_hbm.at[pl.ds(ci * CH, CH)], dst_buf.at[b],
                         sems[b])
        pltpu.async_copy(src_hbm.at[pl.ds(ci * CH, CH)], src_buf.at[b],
                         sems[b])

    def wait_chunk(b):
        pltpu.make_async_copy(dst_hbm.at[pl.ds(0, CH)], dst_buf.at[b],
                              sems[b]).wait()
        pltpu.make_async_copy(src_hbm.at[pl.ds(0, CH)], src_buf.at[b],
                              sems[b]).wait()

    fire_chunk(0, 0)

    def scan_pair(p, cnt):
        for b in range(2):
            ci = p * 2 + b

            @pl.when(ci + 1 < NCHUNK)
            def _():
                fire_chunk(ci + 1, 1 - b)

            wait_chunk(b)

            @plsc.parallel_loop(0, CH // 16, 1, unroll=8, carry=cnt)
            def grp(i, cnt):
                d = dst_buf[b, pl.ds(i * 16, 16)]
                sv = src_buf[b, pl.ds(i * 16, 16)]
                dl = d - lo
                m = dl.astype(jnp.uint32) < jnp.uint32(RNG)
                v = (sv << 9) | (dl & 511)
                plsc.store_compressed(dl_list.at[pl.ds(cnt, 16)], v, mask=m)
                pop = plsc.all_reduce_population_count(m)
                return cnt + pop[0]

            cnt = grp
        return cnt

    cnt = lax.fori_loop(0, NCHUNK // 2, scan_pair, 0)
    dl_list[pl.ds(cnt, 16)] = jnp.full((16,), 511, jnp.int32)
    nit = (cnt + 15) >> 4

    @pl.loop(0, NSUB)
    def sub(s):
        slo = lo + s * SUB
        pltpu.sync_copy(zer_hbm, acc)
        for r in range(SUB + 1):
            den[pl.ds(r * 16, 16)] = jnp.zeros((16,), jnp.float32)

        @plsc.parallel_loop(0, nit, 1, unroll=8, carry=jnp.int32(0))
        def fgrp(i, c):
            pv = dl_list[pl.ds(i * 16, 16)]
            slv = pv >> 9
            t = (pv & 511) - s * SUB
            m = t.astype(jnp.uint32) < jnp.uint32(SUB)
            plsc.store_compressed(dls_buf.at[pl.ds(c, 16)], t, mask=m)
            plsc.store_compressed(sls_buf.at[pl.ds(c, 16)], slv, mask=m)
            pop = plsc.all_reduce_population_count(m)
            return c + pop[0]

        cs = fgrp
        dls_buf[pl.ds(cs, 16)] = jnp.full((16,), SUB, jnp.int32)
        sls_buf[pl.ds(cs, 16)] = jnp.zeros((16,), jnp.int32)
        ng = (cs + (G - 1)) >> 3

        def fire_rows(g, b):
            idx = sls_buf.at[pl.ds(g * G, G)]
            pltpu.async_copy(xpw_hbm.at[idx], rowbuf.at[b], sems[b])

        def wait_rows(b):
            pltpu.make_async_copy(xpw_hbm.at[pl.ds(0, G)], rowbuf.at[b],
                                  sems[b]).wait()

        for pb in range(NBUF - 1):
            @pl.when(pb < ng)
            def _():
                fire_rows(pb, pb)

        def gquad(p, z):
            for b in range(NBUF):
                g = p * NBUF + b

                @pl.when(g < ng)
                def _():
                    @pl.when(g + (NBUF - 1) < ng)
                    def _():
                        fire_rows(g + (NBUF - 1), (b + NBUF - 1) % NBUF)

                    wait_rows(b)
                    dlv = dls_buf[pl.ds(g * G, 16)]
                    for i in range(G):
                        dl = dlv[i]
                        abase = dl * HC
                        asv = rowbuf[b, i, pl.ds(HC, 16)]
                        adv = ad_tab[pl.ds(s * (SUB * 16) + dl * 16, 16)]
                        e = asv + adv
                        el = jnp.where(e > 0.0, e, e * 0.2)
                        ex = jnp.exp(el)
                        plsc.addupdate(den.at[pl.ds(dl * 16, 16)], ex)
                        ibase = i * (H * 16)
                        for h in range(H):
                            bsbuf[pl.ds(ibase + h * 16, 16)] = jnp.full(
                                (16,), ex[h], jnp.float32)
                        @plsc.parallel_loop(0, HC // 16, 1, unroll=8)
                        def _(c):
                            v = rowbuf[b, i, pl.ds(c * 16, 16)]
                            bs = bsbuf[pl.ds(ibase + (c >> 3) * 16, 16)]
                            plsc.addupdate(
                                acc.at[pl.ds(abase + c * 16, 16)], v * bs)
            return z

        lax.fori_loop(0, (ng + NBUF - 1) >> 2, gquad, 0)
        pltpu.sync_copy(acc.at[pl.ds(0, SUB * HC)],
                        acc_hbm.at[pl.ds(slo * HC, SUB * HC)])
        pltpu.sync_copy(den.at[pl.ds(0, SUB * 16)],
                        den_hbm.at[pl.ds(slo * 16, SUB * 16)])


def _sc_edge(dst2, src2, ad_flat, xpw, zer):
    mesh = plsc.VectorSubcoreMesh(core_axis_name="c", subcore_axis_name="s")
    kern = pl.kernel(
        _edge_body,
        out_type=[
            jax.ShapeDtypeStruct((NPAD * HC,), jnp.float32),
            jax.ShapeDtypeStruct((NPAD * 16,), jnp.float32),
        ],
        mesh=mesh,
        compiler_params=_sc_compiler_params(),
        scratch_types=[
            pltpu.VMEM(((RNG + 8) * 16,), jnp.float32),
            pltpu.VMEM((2, CH), jnp.int32),
            pltpu.VMEM((2, CH), jnp.int32),
            pltpu.VMEM((KCAP,), jnp.int32),
            pltpu.VMEM((SCAP,), jnp.int32),
            pltpu.VMEM((SCAP,), jnp.int32),
            pltpu.VMEM((NBUF, G, XW), jnp.float32),
            pltpu.VMEM((G * H * 16,), jnp.float32),
            pltpu.VMEM(((SUB + 1) * HC,), jnp.float32),
            pltpu.VMEM(((SUB + 1) * 16,), jnp.float32),
            pltpu.SemaphoreType.DMA,
            pltpu.SemaphoreType.DMA,
            pltpu.SemaphoreType.DMA,
            pltpu.SemaphoreType.DMA,
        ],
    )
    return kern(dst2, src2, ad_flat, xpw, zer)


BT = 1000
NBT = N // BT


def _t1_body(acc_ref, den_ref, x_ref, w1t_ref, z_ref, s_ref, q_ref):
    i = pl.program_id(0)
    a = acc_ref[...]
    dn = den_ref[...]
    w1t = w1t_ref[...]
    h1 = jnp.zeros((BT, D), jnp.float32)
    for h in range(H):
        yh = lax.dot_general(a[:, h * C:(h + 1) * C], w1t,
                             (((1,), (0,)), ((), ())),
                             preferred_element_type=jnp.float32)
        rh = 1.0 / (dn[:, h:h + 1] + 1e-16)
        h1 = h1 + yh * rh
    z = x_ref[...] + h1
    z_ref[...] = z
    ps = jnp.sum(z, axis=0, keepdims=True)
    pq = jnp.sum(z * z, axis=0, keepdims=True)

    @pl.when(i == 0)
    def _():
        s_ref[...] = jnp.zeros_like(s_ref)
        q_ref[...] = jnp.zeros_like(q_ref)

    s_ref[...] += ps
    q_ref[...] += pq


def _t2_body(z_ref, s_ref, q_ref, g1_ref, b1_ref, w2_ref, b2_ref,
             w3_ref, b3_ref, z2_ref, s2_ref, q2_ref):
    i = pl.program_id(0)
    mean = s_ref[...] / N
    var = q_ref[...] / N - mean * mean
    inv = lax.rsqrt(var + 1e-5)
    zn = (z_ref[...] - mean) * inv * g1_ref[...] + b1_ref[...]
    h2 = lax.dot_general(zn, w2_ref[...], (((1,), (1,)), ((), ())),
                         preferred_element_type=jnp.float32)
    h2 = jnp.maximum(h2 + b2_ref[...], 0.0)
    h3 = lax.dot_general(h2, w3_ref[...], (((1,), (1,)), ((), ())),
                         preferred_element_type=jnp.float32)
    z2 = h3 + b3_ref[...] + zn
    z2_ref[...] = z2
    ps = jnp.sum(z2, axis=0, keepdims=True)
    pq = jnp.sum(z2 * z2, axis=0, keepdims=True)

    @pl.when(i == 0)
    def _():
        s2_ref[...] = jnp.zeros_like(s2_ref)
        q2_ref[...] = jnp.zeros_like(q2_ref)

    s2_ref[...] += ps
    q2_ref[...] += pq


def _t3_body(z2_ref, s2_ref, q2_ref, g2_ref, b2b_ref, o_ref):
    mean = s2_ref[...] / N
    var = q2_ref[...] / N - mean * mean
    inv = lax.rsqrt(var + 1e-5)
    o_ref[...] = (z2_ref[...] - mean) * inv * g2_ref[...] + b2b_ref[...]


def _tail(acc, den, x, w1t, g1, b1, w2, b2, w3, b3, g2, b2b):
    row = lambda i: (i, 0)
    const = lambda i: (0, 0)
    z1, s1, q1 = pl.pallas_call(
        _t1_body,
        grid=(NBT,),
        in_specs=[
            pl.BlockSpec((BT, HC), row),
            pl.BlockSpec((BT, 16), row),
            pl.BlockSpec((BT, D), row),
            pl.BlockSpec((C, D), const),
        ],
        out_specs=[
            pl.BlockSpec((BT, D), row),
            pl.BlockSpec((1, D), const),
            pl.BlockSpec((1, D), const),
        ],
        out_shape=[
            jax.ShapeDtypeStruct((N, D), jnp.float32),
            jax.ShapeDtypeStruct((1, D), jnp.float32),
            jax.ShapeDtypeStruct((1, D), jnp.float32),
        ],
    )(acc, den, x, w1t)
    z2, s2, q2 = pl.pallas_call(
        _t2_body,
        grid=(NBT,),
        in_specs=[
            pl.BlockSpec((BT, D), row),
            pl.BlockSpec((1, D), const),
            pl.BlockSpec((1, D), const),
            pl.BlockSpec((1, D), const),
            pl.BlockSpec((1, D), const),
            pl.BlockSpec((HID, D), const),
            pl.BlockSpec((1, HID), const),
            pl.BlockSpec((D, HID), const),
            pl.BlockSpec((1, D), const),
        ],
        out_specs=[
            pl.BlockSpec((BT, D), row),
            pl.BlockSpec((1, D), const),
            pl.BlockSpec((1, D), const),
        ],
        out_shape=[
            jax.ShapeDtypeStruct((N, D), jnp.float32),
            jax.ShapeDtypeStruct((1, D), jnp.float32),
            jax.ShapeDtypeStruct((1, D), jnp.float32),
        ],
    )(z1, s1, q1, g1, b1, w2, b2, w3, b3)
    out = pl.pallas_call(
        _t3_body,
        grid=(NBT,),
        in_specs=[
            pl.BlockSpec((BT, D), row),
            pl.BlockSpec((1, D), const),
            pl.BlockSpec((1, D), const),
            pl.BlockSpec((1, D), const),
            pl.BlockSpec((1, D), const),
        ],
        out_specs=pl.BlockSpec((BT, D), row),
        out_shape=jax.ShapeDtypeStruct((N, D), jnp.float32),
    )(z2, s2, q2, g2, b2b)
    return out


def kernel(x, edge_index, W_gat, att_src, att_dst, W1, bn1_gamma, bn1_beta,
           W2, b2, W3, b3, bn2_gamma, bn2_beta):
    loop_idx = jnp.arange(N, dtype=edge_index.dtype)
    src2 = jnp.concatenate(
        [edge_index[0], loop_idx,
         jnp.zeros((EPAD - EP,), edge_index.dtype)])
    dst2 = jnp.concatenate(
        [edge_index[1], loop_idx,
         jnp.full((EPAD - EP,), -1, edge_index.dtype)])
    x_pad = jnp.pad(x, ((0, NPAD - N), (0, 0)))

    xpw, a_d = _prep(x_pad, W_gat,
                     att_src.reshape(H, C), att_dst.reshape(H, C))

    zer = jnp.zeros(((SUB + 1) * HC,), jnp.float32)
    acc_flat, den_flat = _sc_edge(dst2, src2, a_d.reshape(-1), xpw, zer)
    acc = acc_flat.reshape(NPAD, HC)
    den = den_flat.reshape(NPAD, 16)

    out = _tail(acc[:N], den[:N], x, W1.T,
                bn1_gamma.reshape(1, D), bn1_beta.reshape(1, D),
                W2, b2.reshape(1, HID), W3, b3.reshape(1, D),
                bn2_gamma.reshape(1, D), bn2_beta.reshape(1, D))
    return out


# speedup vs baseline: 1.0924x; 1.0924x over previous
import dataclasses
import functools

import jax
import jax.numpy as jnp
import numpy as np
from jax import lax
from jax.experimental import pallas as pl
from jax.experimental.pallas import tpu as pltpu
from jax.experimental.pallas import tpu_sc as plsc

N = 10000
E = 320000
D = 128
H = 8
C = 128
HC = H * C
HID = 512

NW = 32
RNG = 320
SUB = 64
NSUB = RNG // SUB
NBUF = 2
NBUF_SHIFT = NBUF.bit_length() - 1
NPAD = NW * RNG
CH = 2048
EP = E + N
NCHUNK = -(-EP // CH)
EPAD = NCHUNK * CH
KCAP = 12288
SCAP = 3072
G = 8
XW = HC + 128
_SPLAT_IDX = tuple(np.full((16,), h, np.int32) for h in range(H))


def _sc_compiler_params():
    cp = pltpu.CompilerParams()
    if "needs_layout_passes" in pltpu.CompilerParams.__dataclass_fields__:
        cp = dataclasses.replace(cp, needs_layout_passes=False)
    return cp


def _prep_body(x_ref, wg_ref, asrc_ref, adst_ref, xpw_ref, ad_ref):
    xb = x_ref[...]
    wg = wg_ref[...]
    xp = lax.dot_general(xb, wg, (((1,), (1,)), ((), ())),
                         preferred_element_type=jnp.float32)
    xpw_ref[:, :HC] = xp
    z8 = jnp.zeros((xb.shape[0], 8), jnp.float32)
    z112 = jnp.zeros((xb.shape[0], 112), jnp.float32)
    a_s = []
    a_d = []
    for h in range(H):
        blk = xp[:, h * C:(h + 1) * C]
        a_s.append(jnp.sum(blk * asrc_ref[h:h + 1, :], axis=1, keepdims=True))
        a_d.append(jnp.sum(blk * adst_ref[h:h + 1, :], axis=1, keepdims=True))
    xpw_ref[:, HC:] = jnp.concatenate(a_s + [z8, z112], axis=1)
    ad_ref[...] = jnp.concatenate(a_d + [z8], axis=1)


def _prep(x_pad, w_gat, asrc, adst):
    BR = 1024
    grid = (NPAD // BR,)
    return pl.pallas_call(
        _prep_body,
        grid=grid,
        in_specs=[
            pl.BlockSpec((BR, D), lambda i: (i, 0)),
            pl.BlockSpec((HC, D), lambda i: (0, 0)),
            pl.BlockSpec((H, C), lambda i: (0, 0)),
            pl.BlockSpec((H, C), lambda i: (0, 0)),
        ],
        out_specs=[
            pl.BlockSpec((BR, XW), lambda i: (i, 0)),
            pl.BlockSpec((BR, 16), lambda i: (i, 0)),
        ],
        out_shape=[
            jax.ShapeDtypeStruct((NPAD, XW), jnp.float32),
            jax.ShapeDtypeStruct((NPAD, 16), jnp.float32),
        ],
    )(x_pad, w_gat, asrc, adst)


def _edge_body(dst_hbm, src_hbm, ad_hbm, xpw_hbm, zer_hbm,
               acc_hbm, den_hbm,
               ad_tab, dst_buf, src_buf, dl_list, dls_buf, sls_buf,
               rowbuf, bsbuf, acc, den, sem_a, sem_b):
    cid = lax.axis_index("c")
    sid = lax.axis_index("s")
    w = sid * 2 + cid
    lo = w * RNG
    sems = (sem_a, sem_b)

    pltpu.sync_copy(ad_hbm.at[pl.ds(lo * 16, RNG * 16)],
                    ad_tab.at[pl.ds(0, RNG * 16)])

    def fire_chunk(ci, b):
        pltpu.async_copy(dst_hbm.at[pl.ds(ci * CH, CH)], dst_buf.at[b],
                         sems[b])
        pltpu.async_copy(src_hbm.at[pl.ds(ci * CH, CH)], src_buf.at[b],
                         sems[b])

    def wait_chunk(b):
        pltpu.make_async_copy(dst_hbm.at[pl.ds(0, CH)], dst_buf.at[b],
                              sems[b]).wait()
        pltpu.make_async_copy(src_hbm.at[pl.ds(0, CH)], src_buf.at[b],
                              sems[b]).wait()

    fire_chunk(0, 0)

    def scan_pair(p, cnt):
        for b in range(2):
            ci = p * 2 + b

            @pl.when(ci + 1 < NCHUNK)
            def _():
                fire_chunk(ci + 1, 1 - b)

            wait_chunk(b)

            @plsc.parallel_loop(0, CH // 16, 1, unroll=8, carry=cnt)
            def grp(i, cnt):
                d = dst_buf[b, pl.ds(i * 16, 16)]
                sv = src_buf[b, pl.ds(i * 16, 16)]
                dl = d - lo
                m = dl.astype(jnp.uint32) < jnp.uint32(RNG)
                v = (sv << 9) | (dl & 511)
                plsc.store_compressed(dl_list.at[pl.ds(cnt, 16)], v, mask=m)
                pop = plsc.all_reduce_population_count(m)
                return cnt + pop[0]

            cnt = grp
        return cnt

    cnt = lax.fori_loop(0, NCHUNK // 2, scan_pair, 0)
    dl_list[pl.ds(cnt, 16)] = jnp.full((16,), 511, jnp.int32)
    nit = (cnt + 15) >> 4

    @pl.loop(0, NSUB)
    def sub(s):
        slo = lo + s * SUB
        pltpu.sync_copy(zer_hbm, acc)
        for r in range(SUB + 1):
            den[pl.ds(r * 16, 16)] = jnp.zeros((16,), jnp.float32)

        @plsc.parallel_loop(0, nit, 1, unroll=8, carry=jnp.int32(0))
        def fgrp(i, c):
            pv = dl_list[pl.ds(i * 16, 16)]
            slv = pv >> 9
            t = (pv & 511) - s * SUB
            m = t.astype(jnp.uint32) < jnp.uint32(SUB)
            plsc.store_compressed(dls_buf.at[pl.ds(c, 16)], t, mask=m)
            plsc.store_compressed(sls_buf.at[pl.ds(c, 16)], slv, mask=m)
            pop = plsc.all_reduce_population_count(m)
            return c + pop[0]

        cs = fgrp
        dls_buf[pl.ds(cs, 16)] = jnp.full((16,), SUB, jnp.int32)
        sls_buf[pl.ds(cs, 16)] = jnp.zeros((16,), jnp.int32)
        ng = (cs + (G - 1)) >> 3

        def fire_rows(g, b):
            idx = sls_buf.at[pl.ds(g * G, G)]
            pltpu.async_copy(xpw_hbm.at[idx], rowbuf.at[b], sems[b])

        def wait_rows(b):
            pltpu.make_async_copy(xpw_hbm.at[pl.ds(0, G)], rowbuf.at[b],
                                  sems[b]).wait()

        for pb in range(NBUF - 1):
            @pl.when(pb < ng)
            def _():
                fire_rows(pb, pb)

        def gquad(p, z):
            for b in range(NBUF):
                g = p * NBUF + b

                @pl.when(g < ng)
                def _():
                    @pl.when(g + (NBUF - 1) < ng)
                    def _():
                        fire_rows(g + (NBUF - 1), (b + NBUF - 1) % NBUF)

                    wait_rows(b)
                    dlv = dls_buf[pl.ds(g * G, 16)]
                    for i in range(G):
                        dl = dlv[i]
                        abase = dl * HC
                        asv = rowbuf[b, i, pl.ds(HC, 16)]
                        adv = ad_tab[pl.ds(s * (SUB * 16) + dl * 16, 16)]
                        e = asv + adv
                        el = jnp.where(e > 0.0, e, e * 0.2)
                        ex = jnp.exp(el)
                        plsc.addupdate(den.at[pl.ds(dl * 16, 16)], ex)
                        ibase = i * (H * 16)
                        for h in range(H):
                            bsbuf[pl.ds(ibase + h * 16, 16)] = jnp.full(
                                (16,), ex[h], jnp.float32)
                        @plsc.parallel_loop(0, HC // 16, 1, unroll=8)
                        def _(c):
                            v = rowbuf[b, i, pl.ds(c * 16, 16)]
                            bs = bsbuf[pl.ds(ibase + (c >> 3) * 16, 16)]
                            plsc.addupdate(
                                acc.at[pl.ds(abase + c * 16, 16)], v * bs)
            return z

        lax.fori_loop(0, (ng + NBUF - 1) >> NBUF_SHIFT, gquad, 0)
        pltpu.sync_copy(acc.at[pl.ds(0, SUB * HC)],
                        acc_hbm.at[pl.ds(slo * HC, SUB * HC)])
        pltpu.sync_copy(den.at[pl.ds(0, SUB * 16)],
                        den_hbm.at[pl.ds(slo * 16, SUB * 16)])


def _sc_edge(dst2, src2, ad_flat, xpw, zer):
    mesh = plsc.VectorSubcoreMesh(core_axis_name="c", subcore_axis_name="s")
    kern = pl.kernel(
        _edge_body,
        out_type=[
            jax.ShapeDtypeStruct((NPAD * HC,), jnp.float32),
            jax.ShapeDtypeStruct((NPAD * 16,), jnp.float32),
        ],
        mesh=mesh,
        compiler_params=_sc_compiler_params(),
        scratch_types=[
            pltpu.VMEM(((RNG + 8) * 16,), jnp.float32),
            pltpu.VMEM((2, CH), jnp.int32),
            pltpu.VMEM((2, CH), jnp.int32),
            pltpu.VMEM((KCAP,), jnp.int32),
            pltpu.VMEM((SCAP,), jnp.int32),
            pltpu.VMEM((SCAP,), jnp.int32),
            pltpu.VMEM((NBUF, G, XW), jnp.float32),
            pltpu.VMEM((G * H * 16,), jnp.float32),
            pltpu.VMEM(((SUB + 1) * HC,), jnp.float32),
            pltpu.VMEM(((SUB + 1) * 16,), jnp.float32),
            pltpu.SemaphoreType.DMA,
            pltpu.SemaphoreType.DMA,
        ],
    )
    return kern(dst2, src2, ad_flat, xpw, zer)


BT = 1000
NBT = N // BT


def _t1_body(acc_ref, den_ref, x_ref, w1t_ref, z_ref, s_ref, q_ref):
    i = pl.program_id(0)
    a = acc_ref[...]
    dn = den_ref[...]
    w1t = w1t_ref[...]
    h1 = jnp.zeros((BT, D), jnp.float32)
    for h in range(H):
        yh = lax.dot_general(a[:, h * C:(h + 1) * C], w1t,
                             (((1,), (0,)), ((), ())),
                             preferred_element_type=jnp.float32)
        rh = 1.0 / (dn[:, h:h + 1] + 1e-16)
        h1 = h1 + yh * rh
    z = x_ref[...] + h1
    z_ref[...] = z
    ps = jnp.sum(z, axis=0, keepdims=True)
    pq = jnp.sum(z * z, axis=0, keepdims=True)

    @pl.when(i == 0)
    def _():
        s_ref[...] = jnp.zeros_like(s_ref)
        q_ref[...] = jnp.zeros_like(q_ref)

    s_ref[...] += ps
    q_ref[...] += pq


def _t2_body(z_ref, s_ref, q_ref, g1_ref, b1_ref, w2_ref, b2_ref,
             w3_ref, b3_ref, z2_ref, s2_ref, q2_ref):
    i = pl.program_id(0)
    mean = s_ref[...] / N
    var = q_ref[...] / N - mean * mean
    inv = lax.rsqrt(var + 1e-5)
    zn = (z_ref[...] - mean) * inv * g1_ref[...] + b1_ref[...]
    h2 = lax.dot_general(zn, w2_ref[...], (((1,), (1,)), ((), ())),
                         preferred_element_type=jnp.float32)
    h2 = jnp.maximum(h2 + b2_ref[...], 0.0)
    h3 = lax.dot_general(h2, w3_ref[...], (((1,), (1,)), ((), ())),
                         preferred_element_type=jnp.float32)
    z2 = h3 + b3_ref[...] + zn
    z2_ref[...] = z2
    ps = jnp.sum(z2, axis=0, keepdims=True)
    pq = jnp.sum(z2 * z2, axis=0, keepdims=True)

    @pl.when(i == 0)
    def _():
        s2_ref[...] = jnp.zeros_like(s2_ref)
        q2_ref[...] = jnp.zeros_like(q2_ref)

    s2_ref[...] += ps
    q2_ref[...] += pq


def _t3_body(z2_ref, s2_ref, q2_ref, g2_ref, b2b_ref, o_ref):
    mean = s2_ref[...] / N
    var = q2_ref[...] / N - mean * mean
    inv = lax.rsqrt(var + 1e-5)
    o_ref[...] = (z2_ref[...] - mean) * inv * g2_ref[...] + b2b_ref[...]


def _tail(acc, den, x, w1t, g1, b1, w2, b2, w3, b3, g2, b2b):
    row = lambda i: (i, 0)
    const = lambda i: (0, 0)
    z1, s1, q1 = pl.pallas_call(
        _t1_body,
        grid=(NBT,),
        in_specs=[
            pl.BlockSpec((BT, HC), row),
            pl.BlockSpec((BT, 16), row),
            pl.BlockSpec((BT, D), row),
            pl.BlockSpec((C, D), const),
        ],
        out_specs=[
            pl.BlockSpec((BT, D), row),
            pl.BlockSpec((1, D), const),
            pl.BlockSpec((1, D), const),
        ],
        out_shape=[
            jax.ShapeDtypeStruct((N, D), jnp.float32),
            jax.ShapeDtypeStruct((1, D), jnp.float32),
            jax.ShapeDtypeStruct((1, D), jnp.float32),
        ],
    )(acc, den, x, w1t)
    z2, s2, q2 = pl.pallas_call(
        _t2_body,
        grid=(NBT,),
        in_specs=[
            pl.BlockSpec((BT, D), row),
            pl.BlockSpec((1, D), const),
            pl.BlockSpec((1, D), const),
            pl.BlockSpec((1, D), const),
            pl.BlockSpec((1, D), const),
            pl.BlockSpec((HID, D), const),
            pl.BlockSpec((1, HID), const),
            pl.BlockSpec((D, HID), const),
            pl.BlockSpec((1, D), const),
        ],
        out_specs=[
            pl.BlockSpec((BT, D), row),
            pl.BlockSpec((1, D), const),
            pl.BlockSpec((1, D), const),
        ],
        out_shape=[
            jax.ShapeDtypeStruct((N, D), jnp.float32),
            jax.ShapeDtypeStruct((1, D), jnp.float32),
            jax.ShapeDtypeStruct((1, D), jnp.float32),
        ],
    )(z1, s1, q1, g1, b1, w2, b2, w3, b3)
    out = pl.pallas_call(
        _t3_body,
        grid=(NBT,),
        in_specs=[
            pl.BlockSpec((BT, D), row),
            pl.BlockSpec((1, D), const),
            pl.BlockSpec((1, D), const),
            pl.BlockSpec((1, D), const),
            pl.BlockSpec((1, D), const),
        ],
        out_specs=pl.BlockSpec((BT, D), row),
        out_shape=jax.ShapeDtypeStruct((N, D), jnp.float32),
    )(z2, s2, q2, g2, b2b)
    return out


def kernel(x, edge_index, W_gat, att_src, att_dst, W1, bn1_gamma, bn1_beta,
           W2, b2, W3, b3, bn2_gamma, bn2_beta):
    loop_idx = jnp.arange(N, dtype=edge_index.dtype)
    src2 = jnp.concatenate(
        [edge_index[0], loop_idx,
         jnp.zeros((EPAD - EP,), edge_index.dtype)])
    dst2 = jnp.concatenate(
        [edge_index[1], loop_idx,
         jnp.full((EPAD - EP,), -1, edge_index.dtype)])
    x_pad = jnp.pad(x, ((0, NPAD - N), (0, 0)))

    xpw, a_d = _prep(x_pad, W_gat,
                     att_src.reshape(H, C), att_dst.reshape(H, C))

    zer = jnp.zeros(((SUB + 1) * HC,), jnp.float32)
    acc_flat, den_flat = _sc_edge(dst2, src2, a_d.reshape(-1), xpw, zer)
    acc = acc_flat.reshape(NPAD, HC)
    den = den_flat.reshape(NPAD, 16)

    out = _tail(acc[:N], den[:N], x, W1.T,
                bn1_gamma.reshape(1, D), bn1_beta.reshape(1, D),
                W2, b2.reshape(1, HID), W3, b3.reshape(1, D),
                bn2_gamma.reshape(1, D), bn2_beta.reshape(1, D))
    return out
